# baseline (device time: 249925 ns/iter reference)
import jax
import jax.numpy as jnp
from jax import lax
from jax.experimental import pallas as pl
from jax.experimental.pallas import tpu as pltpu

N_DEV = 8
N_HOP = N_DEV - 1
_GELU_C = 0.7978845608028654


def _gelu(y):
    return 0.5 * y * (1.0 + jnp.tanh(_GELU_C * (y + 0.044715 * y * y * y)))


def kernel(x, w_mat):
    m_per, k = x.shape
    _, n_per = w_mat.shape
    half = m_per // 2

    xb = x.astype(jnp.bfloat16)
    wb = w_mat.astype(jnp.bfloat16)

    def body(x_ref, w_ref, out_ref, fwd, bwd, fs_sem, fr_sem, bs_sem, br_sem):
        my = lax.axis_index("i")
        right = lax.rem(my + 1, N_DEV)
        left = lax.rem(my + N_DEV - 1, N_DEV)

        barrier = pltpu.get_barrier_semaphore()
        for nbr in (left, right):
            pl.semaphore_signal(
                barrier, inc=1, device_id=(nbr,),
                device_id_type=pl.DeviceIdType.MESH,
            )
        pl.semaphore_wait(barrier, 2)

        fwd[0, :, :] = x_ref[:half, :]
        bwd[0, :, :] = x_ref[half:, :]

        out_ref[pl.ds(my * m_per, m_per), :] = _gelu(
            jnp.dot(x_ref[:, :], w_ref[:, :], preferred_element_type=jnp.float32)
        )

        for h in range(N_HOP):
            s, r = h % 2, (h + 1) % 2
            rf = pltpu.make_async_remote_copy(
                src_ref=fwd.at[s],
                dst_ref=fwd.at[r],
                send_sem=fs_sem.at[s],
                recv_sem=fr_sem.at[r],
                device_id=(right,),
                device_id_type=pl.DeviceIdType.MESH,
            )
            rb = pltpu.make_async_remote_copy(
                src_ref=bwd.at[s],
                dst_ref=bwd.at[r],
                send_sem=bs_sem.at[s],
                recv_sem=br_sem.at[r],
                device_id=(left,),
                device_id_type=pl.DeviceIdType.MESH,
            )
            rf.start()
            rb.start()
            rf.wait()
            rb.wait()

            origin_f = lax.rem(my + N_DEV - h - 1, N_DEV)
            origin_b = lax.rem(my + h + 1, N_DEV)
            out_ref[pl.ds(origin_f * m_per, half), :] = _gelu(
                jnp.dot(fwd[r], w_ref[:, :], preferred_element_type=jnp.float32)
            )
            out_ref[pl.ds(origin_b * m_per + half, half), :] = _gelu(
                jnp.dot(bwd[r], w_ref[:, :], preferred_element_type=jnp.float32)
            )

    return pl.pallas_call(
        body,
        out_shape=jax.ShapeDtypeStruct((N_DEV * m_per, n_per), jnp.float32),
        in_specs=[
            pl.BlockSpec(memory_space=pltpu.VMEM),
            pl.BlockSpec(memory_space=pltpu.VMEM),
        ],
        out_specs=pl.BlockSpec(memory_space=pltpu.VMEM),
        scratch_shapes=[
            pltpu.VMEM((2, half, k), jnp.bfloat16),
            pltpu.VMEM((2, half, k), jnp.bfloat16),
            pltpu.SemaphoreType.DMA((2,)),
            pltpu.SemaphoreType.DMA((2,)),
            pltpu.SemaphoreType.DMA((2,)),
            pltpu.SemaphoreType.DMA((2,)),
        ],
        compiler_params=pltpu.CompilerParams(collective_id=0),
    )(xb, wb)


# device time: 207567 ns/iter; 1.2041x vs baseline; 1.2041x over previous
import jax
import jax.numpy as jnp
from jax import lax
from jax.experimental import pallas as pl
from jax.experimental.pallas import tpu as pltpu

N_DEV = 8
N_HOP = N_DEV - 1
N_SLOT = 4
Q = 2
_GELU_C = 0.7978845608028654


def _gelu(y):
    return 0.5 * y * (1.0 + jnp.tanh(_GELU_C * (y + 0.044715 * y * y * y)))


def kernel(x, w_mat):
    m_per, k = x.shape
    _, n_per = w_mat.shape
    half = m_per // 2
    sub = half // Q

    xb = x.astype(jnp.bfloat16)
    wb = w_mat.astype(jnp.bfloat16)

    def body(x_ref, w_ref, out_ref, fwd, bwd, fs_sem, fr_sem, bs_sem, br_sem):
        my = lax.axis_index("i")
        right = lax.rem(my + 1, N_DEV)
        left = lax.rem(my + N_DEV - 1, N_DEV)

        barrier = pltpu.get_barrier_semaphore()
        for nbr in (left, right):
            pl.semaphore_signal(
                barrier, inc=1, device_id=(nbr,),
                device_id_type=pl.DeviceIdType.MESH,
            )
        pl.semaphore_wait(barrier, 2)

        def make(h, q):
            s, r = h % N_SLOT, (h + 1) % N_SLOT
            rows = pl.ds(q * sub, sub)
            f = pltpu.make_async_remote_copy(
                src_ref=fwd.at[s, rows, :],
                dst_ref=fwd.at[r, rows, :],
                send_sem=fs_sem.at[s, q],
                recv_sem=fr_sem.at[r, q],
                device_id=(right,),
                device_id_type=pl.DeviceIdType.MESH,
            )
            b = pltpu.make_async_remote_copy(
                src_ref=bwd.at[s, rows, :],
                dst_ref=bwd.at[r, rows, :],
                send_sem=bs_sem.at[s, q],
                recv_sem=br_sem.at[r, q],
                device_id=(left,),
                device_id_type=pl.DeviceIdType.MESH,
            )
            return f, b

        fwd[0, :, :] = x_ref[:half, :]
        bwd[0, :, :] = x_ref[half:, :]
        descs = {q: make(0, q) for q in range(Q)}
        for q in range(Q):
            descs[q][0].start()
            descs[q][1].start()

        out_ref[pl.ds(my * m_per, m_per), :] = _gelu(
            jnp.dot(x_ref[:, :], w_ref[:, :], preferred_element_type=jnp.float32)
        )

        for h in range(1, N_HOP + 1):
            s = h % N_SLOT
            if h < N_HOP:
                nxt = {}
                for q in range(Q):
                    descs[q][0].wait()
                    descs[q][1].wait()
                    f, b = make(h, q)
                    f.start()
                    b.start()
                    nxt[q] = (f, b)
            else:
                for q in range(Q):
                    descs[q][0].wait()
                    descs[q][1].wait()
                nxt = None
            origin_f = lax.rem(my + N_DEV - h, N_DEV)
            origin_b = lax.rem(my + h, N_DEV)
            out_ref[pl.ds(origin_f * m_per, half), :] = _gelu(
                jnp.dot(fwd[s], w_ref[:, :], preferred_element_type=jnp.float32)
            )
            out_ref[pl.ds(origin_b * m_per + half, half), :] = _gelu(
                jnp.dot(bwd[s], w_ref[:, :], preferred_element_type=jnp.float32)
            )
            descs = nxt

    return pl.pallas_call(
        body,
        out_shape=jax.ShapeDtypeStruct((N_DEV * m_per, n_per), jnp.float32),
        in_specs=[
            pl.BlockSpec(memory_space=pltpu.VMEM),
            pl.BlockSpec(memory_space=pltpu.VMEM),
        ],
        out_specs=pl.BlockSpec(memory_space=pltpu.VMEM),
        scratch_shapes=[
            pltpu.VMEM((N_SLOT, half, k), jnp.bfloat16),
            pltpu.VMEM((N_SLOT, half, k), jnp.bfloat16),
            pltpu.SemaphoreType.DMA((N_SLOT, Q)),
            pltpu.SemaphoreType.DMA((N_SLOT, Q)),
            pltpu.SemaphoreType.DMA((N_SLOT, Q)),
            pltpu.SemaphoreType.DMA((N_SLOT, Q)),
        ],
        compiler_params=pltpu.CompilerParams(
            collective_id=0, vmem_limit_bytes=100 * 1024 * 1024
        ),
    )(xb, wb)
